# Initial kernel scaffold; baseline (speedup 1.0000x reference)
import jax
import jax.numpy as jnp
from jax.experimental import pallas as pl


def kernel(indicator_ids, pub_type_ids, category_ids, country_ids, periodicity_ids, importance, days_offset, normalized_value, surprise, ma5, identity_table, type_table, category_table, country_table, periodicity_table, imp_W, imp_b, pe, proj_W, proj_b, ln_gamma, ln_beta):
    raise NotImplementedError("write your pallas kernel here")


# speedup vs baseline: 3.1055x; 3.1055x over previous
import functools
import math

import jax
import jax.numpy as jnp
import numpy as np
from jax import lax
from jax.experimental import pallas as pl
from jax.experimental.pallas import tpu as pltpu
from jax.experimental.pallas import tpu_sc as plsc


def _sc_gather(table, idx2d, n_tokens, chunk=1024, grp=128):
    d = table.shape[1]
    info = plsc.get_sparse_core_info()
    nw = info.num_cores * info.num_subcores
    n_per_w = n_tokens // nw
    n_outer = n_per_w // chunk
    n_grp = chunk // grp
    grp_rows_per_w = n_per_w // grp
    mesh = plsc.VectorSubcoreMesh(core_axis_name="c", subcore_axis_name="s")

    @functools.partial(
        pl.kernel,
        mesh=mesh,
        out_type=jax.ShapeDtypeStruct((n_tokens, d), jnp.float32),
        scratch_types=[
            pltpu.VMEM((n_grp, grp), jnp.int32),
            pltpu.VMEM((chunk, d), jnp.float32),
            pltpu.SemaphoreType.DMA,
        ],
        compiler_params=pltpu.CompilerParams(use_tc_tiling_on_sc=False),
    )
    def k(table_hbm, idx_hbm, out_hbm, idx_v, rows_v, sem):
        wid = lax.axis_index("s") * info.num_cores + lax.axis_index("c")
        tok_base = wid * n_per_w
        row_base = wid * grp_rows_per_w

        def body(i, carry):
            pltpu.sync_copy(idx_hbm.at[pl.ds(row_base + i * n_grp, n_grp)], idx_v)
            copies = [
                pltpu.async_copy(
                    table_hbm.at[idx_v.at[j]],
                    rows_v.at[pl.ds(j * grp, grp)],
                    sem,
                )
                for j in range(n_grp)
            ]
            for c in copies:
                c.wait()
            pltpu.sync_copy(rows_v, out_hbm.at[pl.ds(tok_base + i * chunk, chunk)])
            return carry

        lax.fori_loop(0, n_outer, body, 0)

    return k(table, idx2d)


def _tc_body(gath_ref, pub_ref, cat_ref, cnt_ref, per_ref, imp_ref, days_ref,
             nv_ref, sp_ref, ma_ref, stacked_ref, impw_ref, impb_ref, div_ref,
             w1_ref, w2_ref, pb_ref, g_ref, b_ref, out_ref):
    t = gath_ref.shape[0]
    iota32 = lax.broadcasted_iota(jnp.int32, (t, 32), 1)
    oh = ((pub_ref[...] == iota32).astype(jnp.float32)
          + (cat_ref[...] + 6 == iota32).astype(jnp.float32)
          + (cnt_ref[...] + 14 == iota32).astype(jnp.float32)
          + (per_ref[...] + 20 == iota32).astype(jnp.float32))
    small = jnp.dot(oh, stacked_ref[...], preferred_element_type=jnp.float32,
                    precision=lax.Precision.HIGHEST)
    dayc = jnp.clip(jnp.abs(days_ref[...]), 0, 364).astype(jnp.float32)
    angle = dayc * div_ref[...]
    pe_v = jnp.where((iota32 % 2) == 0, jnp.sin(angle), jnp.cos(angle))
    impv = imp_ref[...] * impw_ref[...] + impb_ref[...]
    acc = gath_ref[...] + small + pe_v + impv
    out64 = jnp.dot(acc, w1_ref[...], preferred_element_type=jnp.float32,
                    precision=lax.Precision.HIGHEST)
    out64 = (out64
             + nv_ref[...] * w2_ref[0:1, :]
             + sp_ref[...] * w2_ref[1:2, :]
             + ma_ref[...] * w2_ref[2:3, :]
             + pb_ref[...])
    m = jnp.mean(out64, axis=1, keepdims=True)
    cen = out64 - m
    var = jnp.mean(cen * cen, axis=1, keepdims=True)
    out_ref[...] = cen * lax.rsqrt(var + 1e-5) * g_ref[...] + b_ref[...]


def _tc_fused(gathered, pub, cat, cnt, per, imp, days, nv, sp, ma,
              stacked, impw_row, impb_row, div_row, w1, w2, pb_row, g_row,
              b_row, t=1024):
    n = gathered.shape[0]
    grid = (n // t,)
    tok = lambda width: pl.BlockSpec((t, width), lambda i: (i, 0))
    full = lambda shp: pl.BlockSpec(shp, lambda i: (0,) * len(shp))
    return pl.pallas_call(
        _tc_body,
        grid=grid,
        in_specs=[
            tok(32),
            tok(1), tok(1), tok(1), tok(1),
            tok(1), tok(1),
            tok(1), tok(1), tok(1),
            full((32, 32)),
            full((1, 32)), full((1, 32)),
            full((1, 32)),
            full((32, 64)),
            full((3, 64)),
            full((1, 64)), full((1, 64)), full((1, 64)),
        ],
        out_specs=tok(64),
        out_shape=jax.ShapeDtypeStruct((n, 64), jnp.float32),
        compiler_params=pltpu.CompilerParams(
            dimension_semantics=("arbitrary",),
        ),
    )(gathered, pub, cat, cnt, per, imp, days, nv, sp, ma,
      stacked, impw_row, impb_row, div_row, w1, w2, pb_row, g_row, b_row)


def kernel(indicator_ids, pub_type_ids, category_ids, country_ids,
           periodicity_ids, importance, days_offset, normalized_value,
           surprise, ma5, identity_table, type_table, category_table,
           country_table, periodicity_table, imp_W, imp_b, pe, proj_W,
           proj_b, ln_gamma, ln_beta):
    b, s = indicator_ids.shape
    n = b * s
    d = identity_table.shape[1]

    idx2d = indicator_ids.astype(jnp.int32).reshape(n // 128, 128)
    gathered = _sc_gather(identity_table, idx2d, n)

    col_i = lambda x: x.astype(jnp.int32).reshape(n, 1)
    col_f = lambda x: x.astype(jnp.float32).reshape(n, 1)

    stacked = jnp.concatenate(
        [type_table, category_table, country_table, periodicity_table,
         jnp.zeros((32 - 25, d), jnp.float32)], axis=0)
    div_term = np.exp(np.arange(0, d, 2).astype(np.float32)
                      * (-math.log(10000.0) / d))
    div_row = jnp.asarray(np.repeat(div_term, 2).reshape(1, d))
    w1 = proj_W[:, :d].T
    w2 = proj_W[:, d:].T

    out = _tc_fused(
        gathered,
        col_i(pub_type_ids), col_i(category_ids), col_i(country_ids),
        col_i(periodicity_ids), col_f(importance), col_i(days_offset),
        col_f(normalized_value), col_f(surprise), col_f(ma5),
        stacked, imp_W[:, 0].reshape(1, d), imp_b.reshape(1, d), div_row,
        w1, w2, proj_b.reshape(1, 64), ln_gamma.reshape(1, 64),
        ln_beta.reshape(1, 64))
    return out.reshape(b, s, 64)


# speedup vs baseline: 11.8151x; 3.8046x over previous
import functools
import math

import jax
import jax.numpy as jnp
import numpy as np
from jax import lax
from jax.experimental import pallas as pl
from jax.experimental.pallas import tpu as pltpu
from jax.experimental.pallas import tpu_sc as plsc


def _sc_gather(table, idx2d, n_tokens, chunk=1024, grp=128):
    d = table.shape[1]
    info = plsc.get_sparse_core_info()
    nw = info.num_cores * info.num_subcores
    n_per_w = n_tokens // nw
    n_outer = n_per_w // chunk
    n_grp = chunk // grp
    grp_rows_per_w = n_per_w // grp
    mesh = plsc.VectorSubcoreMesh(core_axis_name="c", subcore_axis_name="s")

    @functools.partial(
        pl.kernel,
        mesh=mesh,
        out_type=jax.ShapeDtypeStruct((n_tokens, d), jnp.float32),
        scratch_types=[
            pltpu.VMEM((n_grp, grp), jnp.int32),
            pltpu.VMEM((chunk, d), jnp.float32),
            pltpu.SemaphoreType.DMA,
        ],
        compiler_params=pltpu.CompilerParams(use_tc_tiling_on_sc=False),
    )
    def k(table_hbm, idx_hbm, out_hbm, idx_v, rows_v, sem):
        wid = lax.axis_index("s") * info.num_cores + lax.axis_index("c")
        tok_base = wid * n_per_w
        row_base = wid * grp_rows_per_w

        def body(i, carry):
            pltpu.sync_copy(idx_hbm.at[pl.ds(row_base + i * n_grp, n_grp)], idx_v)
            copies = [
                pltpu.async_copy(
                    table_hbm.at[idx_v.at[j]],
                    rows_v.at[pl.ds(j * grp, grp)],
                    sem,
                )
                for j in range(n_grp)
            ]
            for c in copies:
                c.wait()
            pltpu.sync_copy(rows_v, out_hbm.at[pl.ds(tok_base + i * chunk, chunk)])
            return carry

        lax.fori_loop(0, n_outer, body, 0)

    return k(table, idx2d)


_ROWS = 8
_LANES = 1024
_TBLK = _ROWS * _LANES


def _tc_body(gath_ref, pub_ref, cat_ref, cnt_ref, per_ref, imp_ref, days_ref,
             nv_ref, sp_ref, ma_ref, wext_ref, w1_ref, impw_ref, div_ref,
             g_ref, b_ref, out_ref):
    gath = gath_ref[...]
    wext = wext_ref[...].astype(jnp.bfloat16)
    w1 = w1_ref[...].astype(jnp.bfloat16)
    iota_c = lax.broadcasted_iota(jnp.int32, (32, _LANES), 0)
    parity_even = (iota_c % 2) == 0
    one = jnp.float32(1.0)
    zero = jnp.float32(0.0)
    for r in range(_ROWS):
        pub = pub_ref[:, r, :]
        cat = cat_ref[:, r, :]
        cnt = cnt_ref[:, r, :]
        per = per_ref[:, r, :]
        oh = (jnp.where(pub == iota_c, one, zero)
              + jnp.where(cat + 6 == iota_c, one, zero)
              + jnp.where(cnt + 14 == iota_c, one, zero)
              + jnp.where(per + 20 == iota_c, one, zero))
        dayc = jnp.clip(jnp.abs(days_ref[:, r, :]), 0, 364).astype(jnp.float32)
        angle = dayc * div_ref[...]
        pe_v = jnp.where(parity_even, jnp.sin(angle), jnp.cos(angle))
        dense = pe_v + imp_ref[:, r, :] * impw_ref[...]
        accT = jnp.concatenate(
            [oh, dense, nv_ref[:, r, :], sp_ref[:, r, :], ma_ref[:, r, :],
             jnp.ones((1, _LANES), jnp.float32)], axis=0)
        out64 = lax.dot_general(
            accT.astype(jnp.bfloat16), wext,
            (((0,), (0,)), ((), ())),
            preferred_element_type=jnp.float32)
        out64 = out64 + jnp.dot(
            gath[r * _LANES:(r + 1) * _LANES, :].astype(jnp.bfloat16), w1,
            preferred_element_type=jnp.float32)
        m = jnp.mean(out64, axis=1, keepdims=True)
        cen = out64 - m
        var = jnp.mean(cen * cen, axis=1, keepdims=True)
        out_ref[r * _LANES:(r + 1) * _LANES, :] = (
            cen * lax.rsqrt(var + 1e-5) * g_ref[...] + b_ref[...])


def _tc_fused(gathered, pub, cat, cnt, per, imp, days, nv, sp, ma,
              wext, w1, impw_col, div_col, g_row, b_row):
    n = gathered.shape[0]
    grid = (n // _TBLK,)
    tok3 = pl.BlockSpec((1, _ROWS, _LANES), lambda i: (i, 0, 0))
    full = lambda shp: pl.BlockSpec(shp, lambda i: (0,) * len(shp))
    return pl.pallas_call(
        _tc_body,
        grid=grid,
        in_specs=[
            pl.BlockSpec((_TBLK, 32), lambda i: (i, 0)),
            tok3, tok3, tok3, tok3,
            tok3, tok3,
            tok3, tok3, tok3,
            full((68, 64)),
            full((32, 64)),
            full((32, 1)),
            full((32, 1)),
            full((1, 64)), full((1, 64)),
        ],
        out_specs=pl.BlockSpec((_TBLK, 64), lambda i: (i, 0)),
        out_shape=jax.ShapeDtypeStruct((n, 64), jnp.float32),
        compiler_params=pltpu.CompilerParams(
            dimension_semantics=("arbitrary",),
        ),
    )(gathered, pub, cat, cnt, per, imp, days, nv, sp, ma,
      wext, w1, impw_col, div_col, g_row, b_row)


def kernel(indicator_ids, pub_type_ids, category_ids, country_ids,
           periodicity_ids, importance, days_offset, normalized_value,
           surprise, ma5, identity_table, type_table, category_table,
           country_table, periodicity_table, imp_W, imp_b, pe, proj_W,
           proj_b, ln_gamma, ln_beta):
    b, s = indicator_ids.shape
    n = b * s
    d = identity_table.shape[1]

    idx2d = indicator_ids.astype(jnp.int32).reshape(n // 128, 128)
    gathered = _sc_gather(identity_table, idx2d, n)

    g = n // _TBLK
    row3_i = lambda x: x.astype(jnp.int32).reshape(g, _ROWS, _LANES)
    row3_f = lambda x: x.astype(jnp.float32).reshape(g, _ROWS, _LANES)

    stacked = jnp.concatenate(
        [type_table, category_table, country_table, periodicity_table,
         jnp.zeros((32 - 25, d), jnp.float32)], axis=0)
    div_term = np.exp(np.arange(0, d, 2).astype(np.float32)
                      * (-math.log(10000.0) / d))
    div_col = jnp.asarray(np.repeat(div_term, 2).reshape(d, 1))
    w1 = proj_W[:, :d].T
    w2 = proj_W[:, d:].T
    hi = lax.Precision.HIGHEST
    stacked_w1 = jnp.dot(stacked, w1, precision=hi)
    const_row = (jnp.dot(imp_b, w1, precision=hi) + proj_b).reshape(1, 64)
    wext = jnp.concatenate([stacked_w1, w1, w2, const_row], axis=0)

    out = _tc_fused(
        gathered,
        row3_i(pub_type_ids), row3_i(category_ids), row3_i(country_ids),
        row3_i(periodicity_ids), row3_f(importance), row3_i(days_offset),
        row3_f(normalized_value), row3_f(surprise), row3_f(ma5),
        wext, w1, imp_W[:, 0].reshape(d, 1), div_col,
        ln_gamma.reshape(1, 64), ln_beta.reshape(1, 64))
    return out.reshape(b, s, 64)
